# Initial kernel scaffold; baseline (speedup 1.0000x reference)
import jax
import jax.numpy as jnp
from jax.experimental import pallas as pl


def kernel(x, params):
    raise NotImplementedError("write your pallas kernel here")


# speedup vs baseline: 1.4204x; 1.4204x over previous
import functools

import jax
import jax.numpy as jnp
import numpy as np
from jax.experimental import pallas as pl
from jax.experimental.pallas import tpu as pltpu

DIM = 96
QK = 96
MLPD = 192
HEADS = 4
NT = 8
GSZ = 128
DQ = QK // HEADS
DV = DIM // HEADS


def _ln(x, g, b, eps=1e-5):
    m = jnp.mean(x, axis=-1, keepdims=True)
    v = jnp.mean((x - m) ** 2, axis=-1, keepdims=True)
    return (x - m) / jnp.sqrt(v + eps) * g + b


def _l2n(x, eps=1e-12):
    n = jnp.linalg.norm(x, axis=-1, keepdims=True)
    return x / jnp.maximum(n, eps)


def _gelu(x):
    return jax.nn.gelu(x, approximate=False)


def _smax(s):
    m = jnp.max(s, axis=-1, keepdims=True)
    e = jnp.exp(s - m)
    return e / jnp.sum(e, axis=-1, keepdims=True)


def _attn_body(q_ref, k1_ref, k2_ref, v1_ref, v2_ref, kg_ref, vg_ref, o_ref):
    q = q_ref[...]
    k = jnp.concatenate([k1_ref[...], k2_ref[...]], axis=0)
    v = jnp.concatenate([v1_ref[...], v2_ref[...]], axis=0)
    kg = kg_ref[...]
    vg = vg_ref[...]
    lane = jax.lax.broadcasted_iota(jnp.int32, (1, QK), 1)
    scale = float(1.0 / np.sqrt(DQ))
    acc = jnp.zeros((GSZ, DIM), jnp.float32)
    for h in range(HEADS):
        msk = (lane >= DQ * h) & (lane < DQ * (h + 1))
        km = jnp.where(msk, k, 0.0)
        s1 = jax.lax.dot_general(q, km, (((1,), (1,)), ((), ())),
                                 preferred_element_type=jnp.float32) * scale
        p1 = _smax(s1)
        vm = jnp.where(msk, v, 0.0)
        acc += jnp.dot(p1, vm, preferred_element_type=jnp.float32)
        kgm = jnp.where(msk, kg, 0.0)
        s2 = jax.lax.dot_general(q, kgm, (((1,), (1,)), ((), ())),
                                 preferred_element_type=jnp.float32) * scale
        p2 = _smax(s2)
        vgm = jnp.where(msk, vg, 0.0)
        acc += jnp.dot(p2, vgm, preferred_element_type=jnp.float32)
    o_ref[...] = acc


def _windowed_attention(q, k_ext, v_ext, kg, vg, ng):
    spec_q = pl.BlockSpec((GSZ, QK), lambda g: (g, 0))
    spec_kv1 = pl.BlockSpec((GSZ, QK), lambda g: (g, 0))
    spec_kv2 = pl.BlockSpec((GSZ, QK), lambda g: (g + 1, 0))
    spec_p = pl.BlockSpec((NT, QK), lambda g: (0, 0))
    return pl.pallas_call(
        _attn_body,
        grid=(ng,),
        in_specs=[spec_q, spec_kv1, spec_kv2, spec_kv1, spec_kv2,
                  spec_p, spec_p],
        out_specs=pl.BlockSpec((GSZ, DIM), lambda g: (g, 0)),
        out_shape=jax.ShapeDtypeStruct((ng * GSZ, DIM), jnp.float32),
    )(q, k_ext, k_ext, v_ext, v_ext, kg, vg)


def kernel(x, params):
    p = params
    b, c, h, w = x.shape
    N = h * w
    assert b == 1 and N % GSZ == 0
    ng = N // GSZ
    xt = x.reshape(c, N).T
    residual = xt
    xn = _ln(xt, p['norm_g'], p['norm_b'])
    embed = _gelu(_ln(xn, p['d_eln_g'], p['d_eln_b']) @ p['d_embed_w'].T
                  + p['d_embed_b'])
    assignment = jax.nn.softmax(embed @ p['d_assign_w'].T, axis=-1)
    proto_content = assignment.T @ xn
    proto_weight = jnp.clip(jnp.sum(assignment, axis=0)[:, None], 1e-6, None)
    prototypes = proto_content / proto_weight
    prototypes = _l2n(_ln(prototypes, p['d_pn_g'], p['d_pn_b']))
    scale = QK ** (-0.5)
    q_proto = (prototypes + p['d_protoq']) @ p['d_rq_w'].T
    k_tokens = embed @ p['d_rk_w'].T
    v_tokens = xn @ p['d_rv_w'].T
    refine_attn = jax.nn.softmax((q_proto @ k_tokens.T) * scale, axis=-1)
    proto_refine = refine_attn @ v_tokens
    gamma = jax.nn.sigmoid(p['d_gate'])
    prototypes = _l2n(_ln(prototypes + gamma * proto_refine,
                          p['d_pn_g'], p['d_pn_b']))
    token_features = _l2n(embed @ p['d_tp_w'].T)
    prototype_features = _l2n(prototypes @ p['d_pp_w'].T)
    scores = jax.nn.softmax((token_features @ prototype_features.T) * scale,
                            axis=-1)
    x_scores = jnp.max(scores, axis=-1)
    belong = jnp.argmax(scores, axis=-1)
    sort_key = belong.astype(jnp.float32) + 0.5 * (1.0 - x_scores)
    sorted_idx = jnp.argsort(sort_key)
    sorted_x = xn[sorted_idx]
    q = sorted_x @ p['i_q_w'].T
    k = sorted_x @ p['i_k_w'].T
    v = sorted_x @ p['i_v_w'].T
    k_ext = jnp.concatenate([k, k[N - GSZ:][::-1]], axis=0)
    v_ext = jnp.concatenate([v, v[N - GSZ:][::-1]], axis=0)
    kg = prototypes @ p['i_pk_w'].T
    vg = prototypes @ p['i_pv_w'].T
    out = _windowed_attention(q, k_ext, v_ext, kg, vg, ng)
    wc = p['c1_w'] @ p['i_proj_w']
    y2 = out @ wc.T
    unsorted = jnp.zeros_like(y2).at[sorted_idx].set(y2)
    xt2 = residual + unsorted
    z = _ln(xt2, p['m_ln_g'], p['m_ln_b'])
    z = z @ p['m_fc1_w'].T + p['m_fc1_b']
    z = _gelu(z)
    zi = z.T.reshape(1, MLPD, h, w)
    zc = jax.lax.conv_general_dilated(
        zi, p['m_dw_w'], (1, 1), [(2, 2), (2, 2)],
        dimension_numbers=('NCHW', 'OIHW', 'NCHW'),
        feature_group_count=MLPD)
    zc = _gelu(zc + p['m_dw_b'][None, :, None, None])
    z = z + zc.reshape(MLPD, N).T
    z = z @ p['m_fc2_w'].T + p['m_fc2_b']
    xt_out = xt2 + z
    return xt_out.T.reshape(b, DIM, h, w)


# speedup vs baseline: 1.7475x; 1.2303x over previous
import functools

import jax
import jax.numpy as jnp
import numpy as np
from jax.experimental import pallas as pl
from jax.experimental.pallas import tpu as pltpu

DIM = 96
QK = 96
MLPD = 192
HEADS = 4
NT = 8
GSZ = 128
DQ = QK // HEADS
TB = 512
NEG = -1e30


def _ln(x, g, b, eps=1e-5):
    m = jnp.mean(x, axis=-1, keepdims=True)
    v = jnp.mean((x - m) ** 2, axis=-1, keepdims=True)
    return (x - m) / jnp.sqrt(v + eps) * g + b


def _l2n(x, eps=1e-12):
    n = jnp.sqrt(jnp.sum(x * x, axis=-1, keepdims=True))
    return x / jnp.maximum(n, eps)


def _erf(z):
    s = jnp.sign(z)
    a = jnp.abs(z)
    t = 1.0 / (1.0 + 0.3275911 * a)
    poly = ((((1.061405429 * t - 1.453152027) * t + 1.421413741) * t
             - 0.284496736) * t + 0.254829592) * t
    return s * (1.0 - poly * jnp.exp(-a * a))


def _gelu(x):
    return 0.5 * x * (1.0 + _erf(x * 0.7071067811865476))


def _smax(s):
    m = jnp.max(s, axis=-1, keepdims=True)
    e = jnp.exp(s - m)
    return e / jnp.sum(e, axis=-1, keepdims=True)


def _dotT(a, b):
    return jax.lax.dot_general(a, b, (((1,), (1,)), ((), ())),
                               preferred_element_type=jnp.float32)


def _dot0(a, b):
    return jax.lax.dot_general(a, b, (((0,), (0,)), ((), ())),
                               preferred_element_type=jnp.float32)


def _full(shape):
    return pl.BlockSpec(shape, lambda *_: tuple(0 for _ in shape))


def _ka_body(xt_ref, ng_ref, nb_ref, eg_ref, eb_ref, ew_ref, ebias_ref,
             aw_ref, xn_ref, em_ref, ps_ref):
    i = pl.program_id(0)
    xt = xt_ref[...]
    xn = _ln(xt, ng_ref[...], nb_ref[...])
    e0 = _ln(xn, eg_ref[...], eb_ref[...])
    embed = _gelu(_dotT(e0, ew_ref[...]) + ebias_ref[...])
    asg = _smax(_dotT(embed, aw_ref[...]))
    xn_ref[...] = xn
    em_ref[...] = embed

    @pl.when(i == 0)
    def _():
        ps_ref[...] = jnp.zeros_like(ps_ref)

    xnx = jnp.concatenate([xn, jnp.ones((TB, 32), jnp.float32)], axis=1)
    ps_ref[...] += _dot0(asg, xnx)


def _kt1_body(ps_ref, pg_ref, pb_ref, pq_ref, rq_ref, pre_ref, qp_ref):
    ps = ps_ref[...]
    content = ps[:, :DIM]
    weight = jnp.clip(ps[:, DIM:DIM + 1], 1e-6, None)
    protos = _l2n(_ln(content / weight, pg_ref[...], pb_ref[...]))
    pre_ref[...] = protos
    qp_ref[...] = _dotT(protos + pq_ref[...], rq_ref[...])


def _kb_body(xn_ref, em_ref, qp_ref, rk_ref, rv_ref, out_ref,
             m_ref, l_ref, acc_ref):
    i = pl.program_id(0)
    nb = pl.num_programs(0)

    @pl.when(i == 0)
    def _():
        m_ref[...] = jnp.full_like(m_ref, NEG)
        l_ref[...] = jnp.zeros_like(l_ref)
        acc_ref[...] = jnp.zeros_like(acc_ref)

    scale = float(QK) ** (-0.5)
    kt = _dotT(em_ref[...], rk_ref[...])
    vt = _dotT(xn_ref[...], rv_ref[...])
    s = _dotT(qp_ref[...], kt) * scale
    m_old = m_ref[...]
    m_new = jnp.maximum(m_old, jnp.max(s, axis=-1, keepdims=True))
    corr = jnp.exp(m_old - m_new)
    p = jnp.exp(s - m_new)
    l_new = l_ref[...] * corr + jnp.sum(p, axis=-1, keepdims=True)
    acc_new = acc_ref[...] * corr + jnp.dot(
        p, vt, preferred_element_type=jnp.float32)
    m_ref[...] = m_new
    l_ref[...] = l_new
    acc_ref[...] = acc_new

    @pl.when(i == nb - 1)
    def _():
        out_ref[...] = acc_new / l_new


def _kt2_body(pre_ref, rf_ref, gate_ref, pg_ref, pb_ref, pp_ref,
              pk_ref, pv_ref, c1_ref, pj_ref,
              pf_ref, kg_ref, vg_ref, wc_ref):
    gamma = jax.nn.sigmoid(gate_ref[...])
    pt = _l2n(_ln(pre_ref[...] + gamma * rf_ref[...],
                  pg_ref[...], pb_ref[...]))
    pf_ref[...] = _l2n(_dotT(pt, pp_ref[...]))
    kg_ref[...] = _dotT(pt, pk_ref[...])
    vg_ref[...] = _dotT(pt, pv_ref[...])
    wc_ref[...] = jnp.dot(c1_ref[...], pj_ref[...],
                          preferred_element_type=jnp.float32)


def _kc_body(xn_ref, em_ref, pf_ref, tp_ref, qw_ref, kw_ref, vw_ref,
             sk_ref, q_ref, k_ref, v_ref):
    scale = float(QK) ** (-0.5)
    xn = xn_ref[...]
    tf = _l2n(_dotT(em_ref[...], tp_ref[...]))
    sm = _smax(_dotT(tf, pf_ref[...]) * scale)
    xs = jnp.max(sm, axis=-1, keepdims=True)
    iota8 = jax.lax.broadcasted_iota(jnp.int32, (TB, NT), 1)
    bel = jnp.min(jnp.where(sm >= xs, iota8, NT), axis=-1, keepdims=True)
    sk_ref[...] = bel.astype(jnp.float32) + 0.5 * (1.0 - xs)
    q_ref[...] = _dotT(xn, qw_ref[...])
    k_ref[...] = _dotT(xn, kw_ref[...])
    v_ref[...] = _dotT(xn, vw_ref[...])


def _attn_body(q_ref, k1_ref, k2_ref, v1_ref, v2_ref, kg_ref, vg_ref,
               wc_ref, o_ref):
    q = q_ref[...]
    k = jnp.concatenate([k1_ref[...], k2_ref[...]], axis=0)
    v = jnp.concatenate([v1_ref[...], v2_ref[...]], axis=0)
    kg = kg_ref[...]
    vg = vg_ref[...]
    lane = jax.lax.broadcasted_iota(jnp.int32, (1, QK), 1)
    scale = float(1.0 / np.sqrt(DQ))
    acc = jnp.zeros((GSZ, DIM), jnp.float32)
    for h in range(HEADS):
        msk = (lane >= DQ * h) & (lane < DQ * (h + 1))
        km = jnp.where(msk, k, 0.0)
        p1 = _smax(_dotT(q, km) * scale)
        acc += jnp.dot(p1, jnp.where(msk, v, 0.0),
                       preferred_element_type=jnp.float32)
        p2 = _smax(_dotT(q, jnp.where(msk, kg, 0.0)) * scale)
        acc += jnp.dot(p2, jnp.where(msk, vg, 0.0),
                       preferred_element_type=jnp.float32)
    o_ref[...] = _dotT(acc, wc_ref[...])


def _kf1_body(xt_ref, y_ref, lg_ref, lb_ref, w1_ref, b1_ref,
              xt2_ref, a_ref):
    xt2 = xt_ref[...] + y_ref[...]
    xt2_ref[...] = xt2
    z = _ln(xt2, lg_ref[...], lb_ref[...])
    a_ref[...] = _gelu(_dotT(z, w1_ref[...]) + b1_ref[...])


def _kf2_body(xt2_ref, ap_ref, ac_ref, an_ref, dwt_ref, db_ref,
              w2_ref, b2_ref, o_ref, W):
    g = pl.program_id(0)
    slab = jnp.concatenate([ap_ref[...], ac_ref[...], an_ref[...]], axis=0)
    nidx = (g * TB + jax.lax.broadcasted_iota(jnp.int32, (TB, 1), 0))
    hh = nidx // W
    ww = nidx - hh * W
    acc = jnp.zeros((TB, MLPD), jnp.float32)
    dwt = dwt_ref[...]
    for dx in range(-2, 3):
        wv = (ww + dx >= 0) & (ww + dx < W)
        part = jnp.zeros((TB, MLPD), jnp.float32)
        for dy in range(-2, 3):
            hv = (hh + dy >= 0) & (hh + dy < W)
            srow = TB + dy * W + dx
            sl = slab[srow:srow + TB, :]
            tap = (dy + 2) * 5 + (dx + 2)
            wt = dwt[tap:tap + 1, :]
            part += jnp.where(hv, sl, 0.0) * wt
        acc += jnp.where(wv, part, 0.0)
    a = ac_ref[...]
    zc = _gelu(acc + db_ref[...])
    z2 = a + zc
    o_ref[...] = xt2_ref[...] + _dotT(z2, w2_ref[...]) + b2_ref[...]


def kernel(x, params):
    p = params
    b, c, h, w = x.shape
    N = h * w
    assert b == 1 and N % TB == 0 and N % GSZ == 0
    nb = N // TB
    ng = N // GSZ
    f32 = jnp.float32

    def r2(v):
        return v.reshape(1, -1)

    xt = x.reshape(c, N).T

    tok = pl.BlockSpec((TB, DIM), lambda i: (i, 0))
    tok2 = pl.BlockSpec((TB, MLPD), lambda i: (i, 0))

    xn, embed, psum = pl.pallas_call(
        _ka_body,
        grid=(nb,),
        in_specs=[tok, _full((1, DIM)), _full((1, DIM)), _full((1, DIM)),
                  _full((1, DIM)), _full((DIM, DIM)), _full((1, DIM)),
                  _full((NT, DIM))],
        out_specs=[tok, tok, pl.BlockSpec((NT, 128), lambda i: (0, 0))],
        out_shape=[jax.ShapeDtypeStruct((N, DIM), f32),
                   jax.ShapeDtypeStruct((N, DIM), f32),
                   jax.ShapeDtypeStruct((NT, 128), f32)],
    )(xt, r2(p['norm_g']), r2(p['norm_b']), r2(p['d_eln_g']),
      r2(p['d_eln_b']), p['d_embed_w'], r2(p['d_embed_b']), p['d_assign_w'])

    pre, qp = pl.pallas_call(
        _kt1_body,
        in_specs=[_full((NT, 128)), _full((1, DIM)), _full((1, DIM)),
                  _full((NT, DIM)), _full((DIM, DIM))],
        out_specs=[_full((NT, DIM)), _full((NT, DIM))],
        out_shape=[jax.ShapeDtypeStruct((NT, DIM), f32),
                   jax.ShapeDtypeStruct((NT, DIM), f32)],
    )(psum, r2(p['d_pn_g']), r2(p['d_pn_b']), p['d_protoq'], p['d_rq_w'])

    refine = pl.pallas_call(
        _kb_body,
        grid=(nb,),
        in_specs=[tok, tok, _full((NT, DIM)), _full((DIM, DIM)),
                  _full((DIM, DIM))],
        out_specs=pl.BlockSpec((NT, DIM), lambda i: (0, 0)),
        out_shape=jax.ShapeDtypeStruct((NT, DIM), f32),
        scratch_shapes=[pltpu.VMEM((NT, 1), f32), pltpu.VMEM((NT, 1), f32),
                        pltpu.VMEM((NT, DIM), f32)],
    )(xn, embed, qp, p['d_rk_w'], p['d_rv_w'])

    pf, kg, vg, wc = pl.pallas_call(
        _kt2_body,
        in_specs=[_full((NT, DIM)), _full((NT, DIM)), _full((1, 1)),
                  _full((1, DIM)), _full((1, DIM)), _full((DIM, DIM)),
                  _full((DIM, DIM)), _full((DIM, DIM)), _full((DIM, DIM)),
                  _full((DIM, DIM))],
        out_specs=[_full((NT, DIM))] * 3 + [_full((DIM, DIM))],
        out_shape=[jax.ShapeDtypeStruct((NT, DIM), f32)] * 3
        + [jax.ShapeDtypeStruct((DIM, DIM), f32)],
    )(pre, refine, p['d_gate'].reshape(1, 1), r2(p['d_pn_g']),
      r2(p['d_pn_b']), p['d_pp_w'], p['i_pk_w'], p['i_pv_w'],
      p['c1_w'], p['i_proj_w'])

    skey, q, k, v = pl.pallas_call(
        _kc_body,
        grid=(nb,),
        in_specs=[tok, tok, _full((NT, DIM)), _full((DIM, DIM)),
                  _full((DIM, DIM)), _full((DIM, DIM)), _full((DIM, DIM))],
        out_specs=[pl.BlockSpec((TB, 1), lambda i: (i, 0)), tok, tok, tok],
        out_shape=[jax.ShapeDtypeStruct((N, 1), f32)]
        + [jax.ShapeDtypeStruct((N, DIM), f32)] * 3,
    )(xn, embed, pf, p['d_tp_w'], p['i_q_w'], p['i_k_w'], p['i_v_w'])

    sorted_idx = jnp.argsort(skey[:, 0])
    idx_ext = jnp.concatenate([sorted_idx, sorted_idx[N - GSZ:][::-1]])
    qg = q[sorted_idx]
    k_ext = k[idx_ext]
    v_ext = v[idx_ext]

    spec_q = pl.BlockSpec((GSZ, QK), lambda g: (g, 0))
    spec_kv1 = pl.BlockSpec((GSZ, QK), lambda g: (g, 0))
    spec_kv2 = pl.BlockSpec((GSZ, QK), lambda g: (g + 1, 0))
    spec_p = pl.BlockSpec((NT, QK), lambda g: (0, 0))
    y2 = pl.pallas_call(
        _attn_body,
        grid=(ng,),
        in_specs=[spec_q, spec_kv1, spec_kv2, spec_kv1, spec_kv2,
                  spec_p, spec_p, _full((DIM, DIM))],
        out_specs=pl.BlockSpec((GSZ, DIM), lambda g: (g, 0)),
        out_shape=jax.ShapeDtypeStruct((N, DIM), f32),
    )(qg, k_ext, k_ext, v_ext, v_ext, kg, vg, wc)

    y2u = jnp.zeros_like(y2).at[sorted_idx].set(y2)

    xt2, a = pl.pallas_call(
        _kf1_body,
        grid=(nb,),
        in_specs=[tok, tok, _full((1, DIM)), _full((1, DIM)),
                  _full((MLPD, DIM)), _full((1, MLPD))],
        out_specs=[tok, tok2],
        out_shape=[jax.ShapeDtypeStruct((N, DIM), f32),
                   jax.ShapeDtypeStruct((N, MLPD), f32)],
    )(xt, y2u, r2(p['m_ln_g']), r2(p['m_ln_b']), p['m_fc1_w'],
      r2(p['m_fc1_b']))

    dwt = p['m_dw_w'].reshape(MLPD, 25).T
    ap = pl.BlockSpec((TB, MLPD), lambda i: (jnp.maximum(i - 1, 0), 0))
    an = pl.BlockSpec((TB, MLPD), lambda i: (jnp.minimum(i + 1, nb - 1), 0))
    out_tok = pl.pallas_call(
        functools.partial(_kf2_body, W=w),
        grid=(nb,),
        in_specs=[tok, ap, tok2, an, _full((25, MLPD)), _full((1, MLPD)),
                  _full((DIM, MLPD)), _full((1, DIM))],
        out_specs=tok,
        out_shape=jax.ShapeDtypeStruct((N, DIM), f32),
    )(xt2, a, a, a, dwt, r2(p['m_dw_b']), p['m_fc2_w'], r2(p['m_fc2_b']))

    return out_tok.T.reshape(b, DIM, h, w)


# speedup vs baseline: 1.8364x; 1.0508x over previous
import functools

import jax
import jax.numpy as jnp
import numpy as np
from jax.experimental import pallas as pl
from jax.experimental.pallas import tpu as pltpu

DIM = 96
QK = 96
MLPD = 192
HEADS = 4
NT = 8
GSZ = 128
DQ = QK // HEADS
TB = 512
NEG = -1e30


def _ln(x, g, b, eps=1e-5):
    m = jnp.mean(x, axis=-1, keepdims=True)
    v = jnp.mean((x - m) ** 2, axis=-1, keepdims=True)
    return (x - m) / jnp.sqrt(v + eps) * g + b


def _l2n(x, eps=1e-12):
    n = jnp.sqrt(jnp.sum(x * x, axis=-1, keepdims=True))
    return x / jnp.maximum(n, eps)


def _erf(z):
    s = jnp.sign(z)
    a = jnp.abs(z)
    t = 1.0 / (1.0 + 0.3275911 * a)
    poly = ((((1.061405429 * t - 1.453152027) * t + 1.421413741) * t
             - 0.284496736) * t + 0.254829592) * t
    return s * (1.0 - poly * jnp.exp(-a * a))


def _gelu(x):
    return 0.5 * x * (1.0 + _erf(x * 0.7071067811865476))


def _smax(s):
    m = jnp.max(s, axis=-1, keepdims=True)
    e = jnp.exp(s - m)
    return e / jnp.sum(e, axis=-1, keepdims=True)


def _dotT(a, b):
    return jax.lax.dot_general(a, b, (((1,), (1,)), ((), ())),
                               preferred_element_type=jnp.float32)


def _dot0(a, b):
    return jax.lax.dot_general(a, b, (((0,), (0,)), ((), ())),
                               preferred_element_type=jnp.float32)


def _full(shape):
    return pl.BlockSpec(shape, lambda *_: tuple(0 for _ in shape))


_SC_NC = 2
_SC_NS = 16
_SC_NW = _SC_NC * _SC_NS


def _sc_gather3(q, k, v, idx, n_rows):
    from jax.experimental.pallas import tpu_sc as plsc
    bpw = n_rows // _SC_NW
    nch, tail = divmod(bpw, 128)
    mesh = plsc.VectorSubcoreMesh(core_axis_name="c", subcore_axis_name="s")
    f32 = jnp.float32
    scratch = [pltpu.VMEM((128,), jnp.int32), pltpu.VMEM((128, 128), f32),
               pltpu.SemaphoreType.DMA]
    if tail:
        scratch += [pltpu.VMEM((tail,), jnp.int32),
                    pltpu.VMEM((tail, 128), f32)]

    @functools.partial(
        pl.kernel, mesh=mesh,
        out_type=[jax.ShapeDtypeStruct((n_rows, 128), f32)] * 3,
        scratch_types=scratch)
    def go(q_hbm, k_hbm, v_hbm, idx_hbm, qo, ko, vo, idx_v, buf, sem,
           *tail_bufs):
        wid = jax.lax.axis_index("s") * _SC_NC + jax.lax.axis_index("c")
        base = wid * bpw

        def move(off, iv, bv, nidx):
            pltpu.sync_copy(idx_hbm.at[pl.ds(off, nidx)], iv)
            for t_hbm, o_hbm in ((q_hbm, qo), (k_hbm, ko), (v_hbm, vo)):
                pltpu.async_copy(t_hbm.at[iv], bv, sem).wait()
                pltpu.sync_copy(bv, o_hbm.at[pl.ds(off, nidx)])

        for j in range(nch):
            move(base + j * 128, idx_v, buf, 128)
        if tail:
            move(base + nch * 128, tail_bufs[0], tail_bufs[1], tail)

    return go(q, k, v, idx)


def _sc_scatter(y, idx, n_rows):
    from jax.experimental.pallas import tpu_sc as plsc
    bpw = n_rows // _SC_NW
    nch, tail = divmod(bpw, 128)
    mesh = plsc.VectorSubcoreMesh(core_axis_name="c", subcore_axis_name="s")
    f32 = jnp.float32
    scratch = [pltpu.VMEM((128,), jnp.int32), pltpu.VMEM((128, 128), f32),
               pltpu.SemaphoreType.DMA]
    if tail:
        scratch += [pltpu.VMEM((tail,), jnp.int32),
                    pltpu.VMEM((tail, 128), f32)]

    @functools.partial(
        pl.kernel, mesh=mesh,
        out_type=jax.ShapeDtypeStruct((n_rows, 128), f32),
        scratch_types=scratch)
    def go(y_hbm, idx_hbm, out_hbm, idx_v, buf, sem, *tail_bufs):
        wid = jax.lax.axis_index("s") * _SC_NC + jax.lax.axis_index("c")
        base = wid * bpw

        def move(off, iv, bv, nidx):
            pltpu.sync_copy(idx_hbm.at[pl.ds(off, nidx)], iv)
            pltpu.sync_copy(y_hbm.at[pl.ds(off, nidx)], bv)
            pltpu.async_copy(bv, out_hbm.at[iv], sem).wait()

        for j in range(nch):
            move(base + j * 128, idx_v, buf, 128)
        if tail:
            move(base + nch * 128, tail_bufs[0], tail_bufs[1], tail)

    return go(y, idx)


def _ka_body(xt_ref, ng_ref, nb_ref, eg_ref, eb_ref, ew_ref, ebias_ref,
             aw_ref, xn_ref, em_ref, ps_ref):
    i = pl.program_id(0)
    xt = xt_ref[...]
    xn = _ln(xt, ng_ref[...], nb_ref[...])
    e0 = _ln(xn, eg_ref[...], eb_ref[...])
    embed = _gelu(_dotT(e0, ew_ref[...]) + ebias_ref[...])
    asg = _smax(_dotT(embed, aw_ref[...]))
    xn_ref[...] = xn
    em_ref[...] = embed

    @pl.when(i == 0)
    def _():
        ps_ref[...] = jnp.zeros_like(ps_ref)

    xnx = jnp.concatenate([xn, jnp.ones((TB, 32), jnp.float32)], axis=1)
    ps_ref[...] += _dot0(asg, xnx)


def _kt1_body(ps_ref, pg_ref, pb_ref, pq_ref, rq_ref, pre_ref, qp_ref):
    ps = ps_ref[...]
    content = ps[:, :DIM]
    weight = jnp.clip(ps[:, DIM:DIM + 1], 1e-6, None)
    protos = _l2n(_ln(content / weight, pg_ref[...], pb_ref[...]))
    pre_ref[...] = protos
    qp_ref[...] = _dotT(protos + pq_ref[...], rq_ref[...])


def _kb_body(xn_ref, em_ref, qp_ref, rk_ref, rv_ref, out_ref,
             m_ref, l_ref, acc_ref):
    i = pl.program_id(0)
    nb = pl.num_programs(0)

    @pl.when(i == 0)
    def _():
        m_ref[...] = jnp.full_like(m_ref, NEG)
        l_ref[...] = jnp.zeros_like(l_ref)
        acc_ref[...] = jnp.zeros_like(acc_ref)

    scale = float(QK) ** (-0.5)
    kt = _dotT(em_ref[...], rk_ref[...])
    vt = _dotT(xn_ref[...], rv_ref[...])
    s = _dotT(qp_ref[...], kt) * scale
    m_old = m_ref[...]
    m_new = jnp.maximum(m_old, jnp.max(s, axis=-1, keepdims=True))
    corr = jnp.exp(m_old - m_new)
    p = jnp.exp(s - m_new)
    l_new = l_ref[...] * corr + jnp.sum(p, axis=-1, keepdims=True)
    acc_new = acc_ref[...] * corr + jnp.dot(
        p, vt, preferred_element_type=jnp.float32)
    m_ref[...] = m_new
    l_ref[...] = l_new
    acc_ref[...] = acc_new

    @pl.when(i == nb - 1)
    def _():
        out_ref[...] = acc_new / l_new


def _kt2_body(pre_ref, rf_ref, gate_ref, pg_ref, pb_ref, pp_ref,
              pk_ref, pv_ref, c1_ref, pj_ref,
              pf_ref, kg_ref, vg_ref, wc_ref):
    gamma = jax.nn.sigmoid(gate_ref[...])
    pt = _l2n(_ln(pre_ref[...] + gamma * rf_ref[...],
                  pg_ref[...], pb_ref[...]))
    pf_ref[...] = _l2n(_dotT(pt, pp_ref[...]))
    kg_ref[...] = _dotT(pt, pk_ref[...])
    vg_ref[...] = _dotT(pt, pv_ref[...])
    wc = jnp.dot(c1_ref[...], pj_ref[...],
                 preferred_element_type=jnp.float32)
    wc = jnp.concatenate([wc, jnp.zeros((DIM, 32), jnp.float32)], axis=1)
    wc_ref[...] = jnp.concatenate(
        [wc, jnp.zeros((32, 128), jnp.float32)], axis=0)


def _kc_body(xn_ref, em_ref, pf_ref, tp_ref, qw_ref, kw_ref, vw_ref,
             sk_ref, q_ref, k_ref, v_ref):
    scale = float(QK) ** (-0.5)
    xn = xn_ref[...]
    tf = _l2n(_dotT(em_ref[...], tp_ref[...]))
    sm = _smax(_dotT(tf, pf_ref[...]) * scale)
    xs = jnp.max(sm, axis=-1, keepdims=True)
    iota8 = jax.lax.broadcasted_iota(jnp.int32, (TB, NT), 1)
    bel = jnp.min(jnp.where(sm >= xs, iota8, NT), axis=-1, keepdims=True)
    sk_ref[...] = bel.astype(jnp.float32) + 0.5 * (1.0 - xs)
    q_ref[...] = _dotT(xn, qw_ref[...])
    k_ref[...] = _dotT(xn, kw_ref[...])
    v_ref[...] = _dotT(xn, vw_ref[...])


def _attn_body(q_ref, k1_ref, k2_ref, v1_ref, v2_ref, kg_ref, vg_ref,
               wc_ref, o_ref):
    q = q_ref[...]
    k = jnp.concatenate([k1_ref[...], k2_ref[...]], axis=0)
    v = jnp.concatenate([v1_ref[...], v2_ref[...]], axis=0)
    kg = kg_ref[...]
    vg = vg_ref[...]
    lane = jax.lax.broadcasted_iota(jnp.int32, (1, 128), 1)
    scale = float(1.0 / np.sqrt(DQ))
    acc = jnp.zeros((GSZ, 128), jnp.float32)
    for h in range(HEADS):
        msk = (lane >= DQ * h) & (lane < DQ * (h + 1))
        km = jnp.where(msk, k, 0.0)
        p1 = _smax(_dotT(q, km) * scale)
        acc += jnp.dot(p1, jnp.where(msk, v, 0.0),
                       preferred_element_type=jnp.float32)
        p2 = _smax(_dotT(q, jnp.where(msk, kg, 0.0)) * scale)
        acc += jnp.dot(p2, jnp.where(msk, vg, 0.0),
                       preferred_element_type=jnp.float32)
    o_ref[...] = _dotT(acc, wc_ref[...])


def _kf1_body(xt_ref, y_ref, lg_ref, lb_ref, w1_ref, b1_ref,
              xt2_ref, a_ref):
    xt2 = xt_ref[...] + y_ref[:, :DIM]
    xt2_ref[...] = xt2
    z = _ln(xt2, lg_ref[...], lb_ref[...])
    a_ref[...] = _gelu(_dotT(z, w1_ref[...]) + b1_ref[...])


def _kf2_body(xt2_ref, ap_ref, ac_ref, an_ref, dwt_ref, db_ref,
              w2_ref, b2_ref, o_ref, W):
    g = pl.program_id(0)
    slab = jnp.concatenate([ap_ref[...], ac_ref[...], an_ref[...]], axis=0)
    nidx = (g * TB + jax.lax.broadcasted_iota(jnp.int32, (TB, 1), 0))
    hh = nidx // W
    ww = nidx - hh * W
    acc = jnp.zeros((TB, MLPD), jnp.float32)
    dwt = dwt_ref[...]
    for dx in range(-2, 3):
        wv = (ww + dx >= 0) & (ww + dx < W)
        part = jnp.zeros((TB, MLPD), jnp.float32)
        for dy in range(-2, 3):
            hv = (hh + dy >= 0) & (hh + dy < W)
            srow = TB + dy * W + dx
            sl = slab[srow:srow + TB, :]
            tap = (dy + 2) * 5 + (dx + 2)
            wt = dwt[tap:tap + 1, :]
            part += jnp.where(hv, sl, 0.0) * wt
        acc += jnp.where(wv, part, 0.0)
    a = ac_ref[...]
    zc = _gelu(acc + db_ref[...])
    z2 = a + zc
    o_ref[...] = xt2_ref[...] + _dotT(z2, w2_ref[...]) + b2_ref[...]


def kernel(x, params):
    p = params
    b, c, h, w = x.shape
    N = h * w
    assert b == 1 and N % TB == 0 and N % GSZ == 0
    nb = N // TB
    ng = N // GSZ
    f32 = jnp.float32

    def r2(v):
        return v.reshape(1, -1)

    def pad32(w):
        return jnp.concatenate(
            [w, jnp.zeros((32, w.shape[1]), f32)], axis=0)

    xt = x.reshape(c, N).T

    tok = pl.BlockSpec((TB, DIM), lambda i: (i, 0))
    tok2 = pl.BlockSpec((TB, MLPD), lambda i: (i, 0))

    xn, embed, psum = pl.pallas_call(
        _ka_body,
        grid=(nb,),
        in_specs=[tok, _full((1, DIM)), _full((1, DIM)), _full((1, DIM)),
                  _full((1, DIM)), _full((DIM, DIM)), _full((1, DIM)),
                  _full((NT, DIM))],
        out_specs=[tok, tok, pl.BlockSpec((NT, 128), lambda i: (0, 0))],
        out_shape=[jax.ShapeDtypeStruct((N, DIM), f32),
                   jax.ShapeDtypeStruct((N, DIM), f32),
                   jax.ShapeDtypeStruct((NT, 128), f32)],
    )(xt, r2(p['norm_g']), r2(p['norm_b']), r2(p['d_eln_g']),
      r2(p['d_eln_b']), p['d_embed_w'], r2(p['d_embed_b']), p['d_assign_w'])

    pre, qp = pl.pallas_call(
        _kt1_body,
        in_specs=[_full((NT, 128)), _full((1, DIM)), _full((1, DIM)),
                  _full((NT, DIM)), _full((DIM, DIM))],
        out_specs=[_full((NT, DIM)), _full((NT, DIM))],
        out_shape=[jax.ShapeDtypeStruct((NT, DIM), f32),
                   jax.ShapeDtypeStruct((NT, DIM), f32)],
    )(psum, r2(p['d_pn_g']), r2(p['d_pn_b']), p['d_protoq'], p['d_rq_w'])

    refine = pl.pallas_call(
        _kb_body,
        grid=(nb,),
        in_specs=[tok, tok, _full((NT, DIM)), _full((DIM, DIM)),
                  _full((DIM, DIM))],
        out_specs=pl.BlockSpec((NT, DIM), lambda i: (0, 0)),
        out_shape=jax.ShapeDtypeStruct((NT, DIM), f32),
        scratch_shapes=[pltpu.VMEM((NT, 1), f32), pltpu.VMEM((NT, 1), f32),
                        pltpu.VMEM((NT, DIM), f32)],
    )(xn, embed, qp, p['d_rk_w'], p['d_rv_w'])

    pf, kg, vg, wc = pl.pallas_call(
        _kt2_body,
        in_specs=[_full((NT, DIM)), _full((NT, DIM)), _full((1, 1)),
                  _full((1, DIM)), _full((1, DIM)), _full((DIM, DIM)),
                  _full((128, DIM)), _full((128, DIM)), _full((DIM, DIM)),
                  _full((DIM, DIM))],
        out_specs=[_full((NT, DIM)), _full((NT, 128)), _full((NT, 128)),
                   _full((128, 128))],
        out_shape=[jax.ShapeDtypeStruct((NT, DIM), f32),
                   jax.ShapeDtypeStruct((NT, 128), f32),
                   jax.ShapeDtypeStruct((NT, 128), f32),
                   jax.ShapeDtypeStruct((128, 128), f32)],
    )(pre, refine, p['d_gate'].reshape(1, 1), r2(p['d_pn_g']),
      r2(p['d_pn_b']), p['d_pp_w'], pad32(p['i_pk_w']),
      pad32(p['i_pv_w']), p['c1_w'], p['i_proj_w'])

    tok128 = pl.BlockSpec((TB, 128), lambda i: (i, 0))
    skey, q, k, v = pl.pallas_call(
        _kc_body,
        grid=(nb,),
        in_specs=[tok, tok, _full((NT, DIM)), _full((DIM, DIM)),
                  _full((128, DIM)), _full((128, DIM)), _full((128, DIM))],
        out_specs=[pl.BlockSpec((TB, 1), lambda i: (i, 0)),
                   tok128, tok128, tok128],
        out_shape=[jax.ShapeDtypeStruct((N, 1), f32)]
        + [jax.ShapeDtypeStruct((N, 128), f32)] * 3,
    )(xn, embed, pf, p['d_tp_w'], pad32(p['i_q_w']), pad32(p['i_k_w']),
      pad32(p['i_v_w']))

    sorted_idx = jnp.argsort(skey[:, 0]).astype(jnp.int32)
    ext = N + GSZ
    n_rows = _SC_NW * (-(-(ext // _SC_NW) // 8) * 8)
    pad = n_rows - ext
    idx_ext = jnp.concatenate(
        [sorted_idx, sorted_idx[N - GSZ:][::-1],
         jnp.arange(pad, dtype=jnp.int32)])
    qg, k_ext, v_ext = _sc_gather3(q, k, v, idx_ext, n_rows)

    spec_q = pl.BlockSpec((GSZ, 128), lambda g: (g, 0))
    spec_kv1 = pl.BlockSpec((GSZ, 128), lambda g: (g, 0))
    spec_kv2 = pl.BlockSpec((GSZ, 128), lambda g: (g + 1, 0))
    spec_p = pl.BlockSpec((NT, 128), lambda g: (0, 0))
    y2 = pl.pallas_call(
        _attn_body,
        grid=(ng,),
        in_specs=[spec_q, spec_kv1, spec_kv2, spec_kv1, spec_kv2,
                  spec_p, spec_p, _full((128, 128))],
        out_specs=pl.BlockSpec((GSZ, 128), lambda g: (g, 0)),
        out_shape=jax.ShapeDtypeStruct((N, 128), f32),
    )(qg, k_ext, k_ext, v_ext, v_ext, kg, vg, wc)

    y2u = _sc_scatter(y2, sorted_idx, N)

    xt2, a = pl.pallas_call(
        _kf1_body,
        grid=(nb,),
        in_specs=[tok, tok128, _full((1, DIM)), _full((1, DIM)),
                  _full((MLPD, DIM)), _full((1, MLPD))],
        out_specs=[tok, tok2],
        out_shape=[jax.ShapeDtypeStruct((N, DIM), f32),
                   jax.ShapeDtypeStruct((N, MLPD), f32)],
    )(xt, y2u, r2(p['m_ln_g']), r2(p['m_ln_b']), p['m_fc1_w'],
      r2(p['m_fc1_b']))

    dwt = p['m_dw_w'].reshape(MLPD, 25).T
    ap = pl.BlockSpec((TB, MLPD), lambda i: (jnp.maximum(i - 1, 0), 0))
    an = pl.BlockSpec((TB, MLPD), lambda i: (jnp.minimum(i + 1, nb - 1), 0))
    out_tok = pl.pallas_call(
        functools.partial(_kf2_body, W=w),
        grid=(nb,),
        in_specs=[tok, ap, tok2, an, _full((25, MLPD)), _full((1, MLPD)),
                  _full((DIM, MLPD)), _full((1, DIM))],
        out_specs=tok,
        out_shape=jax.ShapeDtypeStruct((N, DIM), f32),
    )(xt2, a, a, a, dwt, r2(p['m_dw_b']), p['m_fc2_w'], r2(p['m_fc2_b']))

    return out_tok.T.reshape(b, DIM, h, w)


# speedup vs baseline: 2.0740x; 1.1294x over previous
import functools

import jax
import jax.numpy as jnp
import numpy as np
from jax.experimental import pallas as pl
from jax.experimental.pallas import tpu as pltpu

DIM = 96
QK = 96
MLPD = 192
HEADS = 4
NT = 8
GSZ = 128
DQ = QK // HEADS
TB = 512
NEG = -1e30


def _ln(x, g, b, eps=1e-5):
    m = jnp.mean(x, axis=-1, keepdims=True)
    v = jnp.mean((x - m) ** 2, axis=-1, keepdims=True)
    return (x - m) / jnp.sqrt(v + eps) * g + b


def _l2n(x, eps=1e-12):
    n = jnp.sqrt(jnp.sum(x * x, axis=-1, keepdims=True))
    return x / jnp.maximum(n, eps)


def _erf(z):
    s = jnp.sign(z)
    a = jnp.abs(z)
    t = 1.0 / (1.0 + 0.3275911 * a)
    poly = ((((1.061405429 * t - 1.453152027) * t + 1.421413741) * t
             - 0.284496736) * t + 0.254829592) * t
    return s * (1.0 - poly * jnp.exp(-a * a))


def _gelu(x):
    return 0.5 * x * (1.0 + _erf(x * 0.7071067811865476))


def _smax(s):
    m = jnp.max(s, axis=-1, keepdims=True)
    e = jnp.exp(s - m)
    return e / jnp.sum(e, axis=-1, keepdims=True)


def _dotT(a, b):
    return jax.lax.dot_general(a, b, (((1,), (1,)), ((), ())),
                               preferred_element_type=jnp.float32)


def _dot0(a, b):
    return jax.lax.dot_general(a, b, (((0,), (0,)), ((), ())),
                               preferred_element_type=jnp.float32)


def _full(shape):
    return pl.BlockSpec(shape, lambda *_: tuple(0 for _ in shape))


_SC_NC = 2
_SC_NS = 16
_SC_NW = _SC_NC * _SC_NS


def _sc_gather3(q, k, v, idx, n_rows):
    from jax.experimental.pallas import tpu_sc as plsc
    bpw = n_rows // _SC_NW
    nch, tail = divmod(bpw, 128)
    mesh = plsc.VectorSubcoreMesh(core_axis_name="c", subcore_axis_name="s")
    f32 = jnp.float32
    scratch = [pltpu.VMEM((128,), jnp.int32), pltpu.VMEM((128, 128), f32),
               pltpu.SemaphoreType.DMA]
    if tail:
        scratch += [pltpu.VMEM((tail,), jnp.int32),
                    pltpu.VMEM((tail, 128), f32)]

    @functools.partial(
        pl.kernel, mesh=mesh,
        out_type=[jax.ShapeDtypeStruct((n_rows, 128), f32)] * 3,
        scratch_types=scratch)
    def go(q_hbm, k_hbm, v_hbm, idx_hbm, qo, ko, vo, idx_v, buf, sem,
           *tail_bufs):
        wid = jax.lax.axis_index("s") * _SC_NC + jax.lax.axis_index("c")
        base = wid * bpw

        def move(off, iv, bv, nidx):
            pltpu.sync_copy(idx_hbm.at[pl.ds(off, nidx)], iv)
            for t_hbm, o_hbm in ((q_hbm, qo), (k_hbm, ko), (v_hbm, vo)):
                pltpu.async_copy(t_hbm.at[iv], bv, sem).wait()
                pltpu.sync_copy(bv, o_hbm.at[pl.ds(off, nidx)])

        for j in range(nch):
            move(base + j * 128, idx_v, buf, 128)
        if tail:
            move(base + nch * 128, tail_bufs[0], tail_bufs[1], tail)

    return go(q, k, v, idx)


def _sc_scatter(y, idx, n_rows):
    from jax.experimental.pallas import tpu_sc as plsc
    bpw = n_rows // _SC_NW
    nch, tail = divmod(bpw, 128)
    mesh = plsc.VectorSubcoreMesh(core_axis_name="c", subcore_axis_name="s")
    f32 = jnp.float32
    scratch = [pltpu.VMEM((128,), jnp.int32), pltpu.VMEM((128, 128), f32),
               pltpu.SemaphoreType.DMA]
    if tail:
        scratch += [pltpu.VMEM((tail,), jnp.int32),
                    pltpu.VMEM((tail, 128), f32)]

    @functools.partial(
        pl.kernel, mesh=mesh,
        out_type=jax.ShapeDtypeStruct((n_rows, 128), f32),
        scratch_types=scratch)
    def go(y_hbm, idx_hbm, out_hbm, idx_v, buf, sem, *tail_bufs):
        wid = jax.lax.axis_index("s") * _SC_NC + jax.lax.axis_index("c")
        base = wid * bpw

        def move(off, iv, bv, nidx):
            pltpu.sync_copy(idx_hbm.at[pl.ds(off, nidx)], iv)
            pltpu.sync_copy(y_hbm.at[pl.ds(off, nidx)], bv)
            pltpu.async_copy(bv, out_hbm.at[iv], sem).wait()

        for j in range(nch):
            move(base + j * 128, idx_v, buf, 128)
        if tail:
            move(base + nch * 128, tail_bufs[0], tail_bufs[1], tail)

    return go(y, idx)


_SR = 512
_SL = 128
_SN = _SR * _SL


def _lex_lt(ka, va, kb, vb):
    return (ka < kb) | ((ka == kb) & (va < vb))


def _bitonic_body(k_ref, v_ref, ko_ref, vo_ref):
    keys = k_ref[...]
    vals = v_ref[...]
    row = jax.lax.broadcasted_iota(jnp.int32, (_SR, 1), 0)
    lane = jax.lax.broadcasted_iota(jnp.int32, (1, _SL), 1)

    kk = 2
    while kk <= _SN:
        j = kk // 2
        while j >= 1:
            if j >= _SL:
                jr = j // _SL
                g = _SR // (2 * jr)
                k4 = keys.reshape(g, 2, jr, _SL)
                pk = jnp.concatenate([k4[:, 1:2], k4[:, 0:1]], axis=1)
                pk = pk.reshape(_SR, _SL)
                v4 = vals.reshape(g, 2, jr, _SL)
                pv = jnp.concatenate([v4[:, 1:2], v4[:, 0:1]], axis=1)
                pv = pv.reshape(_SR, _SL)
                upper = (row & jr) == 0
            else:
                upper = (lane & j) == 0
                pk = jnp.where(
                    upper,
                    jnp.concatenate([keys[:, j:], keys[:, :j]], axis=1),
                    jnp.concatenate([keys[:, _SL - j:], keys[:, :_SL - j]],
                                    axis=1))
                pv = jnp.where(
                    upper,
                    jnp.concatenate([vals[:, j:], vals[:, :j]], axis=1),
                    jnp.concatenate([vals[:, _SL - j:], vals[:, :_SL - j]],
                                    axis=1))
            bk = kk.bit_length() - 1
            if bk >= 16:
                asc = jnp.full((1, 1), True)
            elif bk >= 7:
                asc = ((row >> (bk - 7)) & 1) == 0
            else:
                asc = (lane & kk) == 0
            s = _lex_lt(pk, pv, keys, vals)
            take = (s != upper) != asc
            keys = jnp.where(take, pk, keys)
            vals = jnp.where(take, pv, vals)
            j //= 2
        kk *= 2
    ko_ref[...] = keys
    vo_ref[...] = vals


def _tc_argsort(skey, n):
    pad = jnp.full((_SN - n,), 1e9, jnp.float32)
    keys2d = jnp.concatenate([skey, pad]).reshape(_SR, _SL)
    vals2d = jnp.arange(_SN, dtype=jnp.float32).reshape(_SR, _SL)
    _, vo = pl.pallas_call(
        _bitonic_body,
        out_shape=[jax.ShapeDtypeStruct((_SR, _SL), jnp.float32)] * 2,
    )(keys2d, vals2d)
    return vo.reshape(-1)[:n].astype(jnp.int32)


def _ka_body(xf_ref, ng_ref, nb_ref, eg_ref, eb_ref, ew_ref, ebias_ref,
             aw_ref, xt_ref, xn_ref, em_ref, ps_ref):
    i = pl.program_id(0)
    xt = jnp.transpose(xf_ref[...])
    xt_ref[...] = xt
    xn = _ln(xt, ng_ref[...], nb_ref[...])
    e0 = _ln(xn, eg_ref[...], eb_ref[...])
    embed = _gelu(_dotT(e0, ew_ref[...]) + ebias_ref[...])
    asg = _smax(_dotT(embed, aw_ref[...]))
    xn_ref[...] = xn
    em_ref[...] = embed

    @pl.when(i == 0)
    def _():
        ps_ref[...] = jnp.zeros_like(ps_ref)

    xnx = jnp.concatenate([xn, jnp.ones((TB, 32), jnp.float32)], axis=1)
    ps_ref[...] += _dot0(asg, xnx)


def _kt1_body(ps_ref, pg_ref, pb_ref, pq_ref, rq_ref, pre_ref, qp_ref):
    ps = ps_ref[...]
    content = ps[:, :DIM]
    weight = jnp.clip(ps[:, DIM:DIM + 1], 1e-6, None)
    protos = _l2n(_ln(content / weight, pg_ref[...], pb_ref[...]))
    pre_ref[...] = protos
    qp_ref[...] = _dotT(protos + pq_ref[...], rq_ref[...])


def _kb_body(xn_ref, em_ref, qp_ref, rk_ref, rv_ref, out_ref,
             m_ref, l_ref, acc_ref):
    i = pl.program_id(0)
    nb = pl.num_programs(0)

    @pl.when(i == 0)
    def _():
        m_ref[...] = jnp.full_like(m_ref, NEG)
        l_ref[...] = jnp.zeros_like(l_ref)
        acc_ref[...] = jnp.zeros_like(acc_ref)

    scale = float(QK) ** (-0.5)
    kt = _dotT(em_ref[...], rk_ref[...])
    vt = _dotT(xn_ref[...], rv_ref[...])
    s = _dotT(qp_ref[...], kt) * scale
    m_old = m_ref[...]
    m_new = jnp.maximum(m_old, jnp.max(s, axis=-1, keepdims=True))
    corr = jnp.exp(m_old - m_new)
    p = jnp.exp(s - m_new)
    l_new = l_ref[...] * corr + jnp.sum(p, axis=-1, keepdims=True)
    acc_new = acc_ref[...] * corr + jnp.dot(
        p, vt, preferred_element_type=jnp.float32)
    m_ref[...] = m_new
    l_ref[...] = l_new
    acc_ref[...] = acc_new

    @pl.when(i == nb - 1)
    def _():
        out_ref[...] = acc_new / l_new


def _kt2_body(pre_ref, rf_ref, gate_ref, pg_ref, pb_ref, pp_ref,
              pk_ref, pv_ref, c1_ref, pj_ref,
              pf_ref, kg_ref, vg_ref, wc_ref):
    gamma = jax.nn.sigmoid(gate_ref[...])
    pt = _l2n(_ln(pre_ref[...] + gamma * rf_ref[...],
                  pg_ref[...], pb_ref[...]))
    pf_ref[...] = _l2n(_dotT(pt, pp_ref[...]))
    kg_ref[...] = _dotT(pt, pk_ref[...])
    vg_ref[...] = _dotT(pt, pv_ref[...])
    wc = jnp.dot(c1_ref[...], pj_ref[...],
                 preferred_element_type=jnp.float32)
    wc = jnp.concatenate([wc, jnp.zeros((DIM, 32), jnp.float32)], axis=1)
    wc_ref[...] = jnp.concatenate(
        [wc, jnp.zeros((32, 128), jnp.float32)], axis=0)


def _kc_body(xn_ref, em_ref, pf_ref, tp_ref, qw_ref, kw_ref, vw_ref,
             sk_ref, q_ref, k_ref, v_ref):
    scale = float(QK) ** (-0.5)
    xn = xn_ref[...]
    tf = _l2n(_dotT(em_ref[...], tp_ref[...]))
    sm = _smax(_dotT(tf, pf_ref[...]) * scale)
    xs = jnp.max(sm, axis=-1, keepdims=True)
    iota8 = jax.lax.broadcasted_iota(jnp.int32, (TB, NT), 1)
    bel = jnp.min(jnp.where(sm >= xs, iota8, NT), axis=-1, keepdims=True)
    sk_ref[...] = bel.astype(jnp.float32) + 0.5 * (1.0 - xs)
    q_ref[...] = _dotT(xn, qw_ref[...])
    k_ref[...] = _dotT(xn, kw_ref[...])
    v_ref[...] = _dotT(xn, vw_ref[...])


def _attn_body(q_ref, k1_ref, k2_ref, v1_ref, v2_ref, kg_ref, vg_ref,
               wc_ref, o_ref):
    q = q_ref[...]
    k = jnp.concatenate([k1_ref[...], k2_ref[...]], axis=0)
    v = jnp.concatenate([v1_ref[...], v2_ref[...]], axis=0)
    kg = kg_ref[...]
    vg = vg_ref[...]
    lane = jax.lax.broadcasted_iota(jnp.int32, (1, 128), 1)
    scale = float(1.0 / np.sqrt(DQ))
    acc = jnp.zeros((GSZ, 128), jnp.float32)
    for h in range(HEADS):
        msk = (lane >= DQ * h) & (lane < DQ * (h + 1))
        km = jnp.where(msk, k, 0.0)
        p1 = _smax(_dotT(q, km) * scale)
        acc += jnp.dot(p1, jnp.where(msk, v, 0.0),
                       preferred_element_type=jnp.float32)
        p2 = _smax(_dotT(q, jnp.where(msk, kg, 0.0)) * scale)
        acc += jnp.dot(p2, jnp.where(msk, vg, 0.0),
                       preferred_element_type=jnp.float32)
    o_ref[...] = _dotT(acc, wc_ref[...])


def _kf1_body(xt_ref, y_ref, lg_ref, lb_ref, w1_ref, b1_ref,
              xt2_ref, a_ref):
    xt2 = xt_ref[...] + y_ref[:, :DIM]
    xt2_ref[...] = xt2
    z = _ln(xt2, lg_ref[...], lb_ref[...])
    a_ref[...] = _gelu(_dotT(z, w1_ref[...]) + b1_ref[...])


def _kf2_body(xt2_ref, ap_ref, ac_ref, an_ref, dwt_ref, db_ref,
              w2_ref, b2_ref, o_ref, W):
    g = pl.program_id(0)
    slab = jnp.concatenate([ap_ref[...], ac_ref[...], an_ref[...]], axis=0)
    nidx = (g * TB + jax.lax.broadcasted_iota(jnp.int32, (TB, 1), 0))
    hh = nidx // W
    ww = nidx - hh * W
    acc = jnp.zeros((TB, MLPD), jnp.float32)
    dwt = dwt_ref[...]
    for dx in range(-2, 3):
        wv = (ww + dx >= 0) & (ww + dx < W)
        part = jnp.zeros((TB, MLPD), jnp.float32)
        for dy in range(-2, 3):
            hv = (hh + dy >= 0) & (hh + dy < W)
            srow = TB + dy * W + dx
            sl = slab[srow:srow + TB, :]
            tap = (dy + 2) * 5 + (dx + 2)
            wt = dwt[tap:tap + 1, :]
            part += jnp.where(hv, sl, 0.0) * wt
        acc += jnp.where(wv, part, 0.0)
    a = ac_ref[...]
    zc = _gelu(acc + db_ref[...])
    z2 = a + zc
    res = xt2_ref[...] + _dotT(z2, w2_ref[...]) + b2_ref[...]
    o_ref[...] = jnp.transpose(res)


def kernel(x, params):
    p = params
    b, c, h, w = x.shape
    N = h * w
    assert b == 1 and N % TB == 0 and N % GSZ == 0
    nb = N // TB
    ng = N // GSZ
    f32 = jnp.float32

    def r2(v):
        return v.reshape(1, -1)

    def pad32(w):
        return jnp.concatenate(
            [w, jnp.zeros((32, w.shape[1]), f32)], axis=0)

    xf = x.reshape(c, N)

    tok = pl.BlockSpec((TB, DIM), lambda i: (i, 0))
    tok2 = pl.BlockSpec((TB, MLPD), lambda i: (i, 0))
    fmaj = pl.BlockSpec((DIM, TB), lambda i: (0, i))

    xt, xn, embed, psum = pl.pallas_call(
        _ka_body,
        grid=(nb,),
        in_specs=[fmaj, _full((1, DIM)), _full((1, DIM)), _full((1, DIM)),
                  _full((1, DIM)), _full((DIM, DIM)), _full((1, DIM)),
                  _full((NT, DIM))],
        out_specs=[tok, tok, tok, pl.BlockSpec((NT, 128), lambda i: (0, 0))],
        out_shape=[jax.ShapeDtypeStruct((N, DIM), f32),
                   jax.ShapeDtypeStruct((N, DIM), f32),
                   jax.ShapeDtypeStruct((N, DIM), f32),
                   jax.ShapeDtypeStruct((NT, 128), f32)],
    )(xf, r2(p['norm_g']), r2(p['norm_b']), r2(p['d_eln_g']),
      r2(p['d_eln_b']), p['d_embed_w'], r2(p['d_embed_b']), p['d_assign_w'])

    pre, qp = pl.pallas_call(
        _kt1_body,
        in_specs=[_full((NT, 128)), _full((1, DIM)), _full((1, DIM)),
                  _full((NT, DIM)), _full((DIM, DIM))],
        out_specs=[_full((NT, DIM)), _full((NT, DIM))],
        out_shape=[jax.ShapeDtypeStruct((NT, DIM), f32),
                   jax.ShapeDtypeStruct((NT, DIM), f32)],
    )(psum, r2(p['d_pn_g']), r2(p['d_pn_b']), p['d_protoq'], p['d_rq_w'])

    refine = pl.pallas_call(
        _kb_body,
        grid=(nb,),
        in_specs=[tok, tok, _full((NT, DIM)), _full((DIM, DIM)),
                  _full((DIM, DIM))],
        out_specs=pl.BlockSpec((NT, DIM), lambda i: (0, 0)),
        out_shape=jax.ShapeDtypeStruct((NT, DIM), f32),
        scratch_shapes=[pltpu.VMEM((NT, 1), f32), pltpu.VMEM((NT, 1), f32),
                        pltpu.VMEM((NT, DIM), f32)],
    )(xn, embed, qp, p['d_rk_w'], p['d_rv_w'])

    pf, kg, vg, wc = pl.pallas_call(
        _kt2_body,
        in_specs=[_full((NT, DIM)), _full((NT, DIM)), _full((1, 1)),
                  _full((1, DIM)), _full((1, DIM)), _full((DIM, DIM)),
                  _full((128, DIM)), _full((128, DIM)), _full((DIM, DIM)),
                  _full((DIM, DIM))],
        out_specs=[_full((NT, DIM)), _full((NT, 128)), _full((NT, 128)),
                   _full((128, 128))],
        out_shape=[jax.ShapeDtypeStruct((NT, DIM), f32),
                   jax.ShapeDtypeStruct((NT, 128), f32),
                   jax.ShapeDtypeStruct((NT, 128), f32),
                   jax.ShapeDtypeStruct((128, 128), f32)],
    )(pre, refine, p['d_gate'].reshape(1, 1), r2(p['d_pn_g']),
      r2(p['d_pn_b']), p['d_pp_w'], pad32(p['i_pk_w']),
      pad32(p['i_pv_w']), p['c1_w'], p['i_proj_w'])

    tok128 = pl.BlockSpec((TB, 128), lambda i: (i, 0))
    skey, q, k, v = pl.pallas_call(
        _kc_body,
        grid=(nb,),
        in_specs=[tok, tok, _full((NT, DIM)), _full((DIM, DIM)),
                  _full((128, DIM)), _full((128, DIM)), _full((128, DIM))],
        out_specs=[pl.BlockSpec((TB, 1), lambda i: (i, 0)),
                   tok128, tok128, tok128],
        out_shape=[jax.ShapeDtypeStruct((N, 1), f32)]
        + [jax.ShapeDtypeStruct((N, 128), f32)] * 3,
    )(xn, embed, pf, p['d_tp_w'], pad32(p['i_q_w']), pad32(p['i_k_w']),
      pad32(p['i_v_w']))

    sorted_idx = _tc_argsort(skey[:, 0], N)
    ext = N + GSZ
    n_rows = _SC_NW * (-(-(ext // _SC_NW) // 8) * 8)
    pad = n_rows - ext
    idx_ext = jnp.concatenate(
        [sorted_idx, sorted_idx[N - GSZ:][::-1],
         jnp.arange(pad, dtype=jnp.int32)])
    qg, k_ext, v_ext = _sc_gather3(q, k, v, idx_ext, n_rows)

    spec_q = pl.BlockSpec((GSZ, 128), lambda g: (g, 0))
    spec_kv1 = pl.BlockSpec((GSZ, 128), lambda g: (g, 0))
    spec_kv2 = pl.BlockSpec((GSZ, 128), lambda g: (g + 1, 0))
    spec_p = pl.BlockSpec((NT, 128), lambda g: (0, 0))
    y2 = pl.pallas_call(
        _attn_body,
        grid=(ng,),
        in_specs=[spec_q, spec_kv1, spec_kv2, spec_kv1, spec_kv2,
                  spec_p, spec_p, _full((128, 128))],
        out_specs=pl.BlockSpec((GSZ, 128), lambda g: (g, 0)),
        out_shape=jax.ShapeDtypeStruct((N, 128), f32),
    )(qg, k_ext, k_ext, v_ext, v_ext, kg, vg, wc)

    y2u = _sc_scatter(y2, sorted_idx, N)

    xt2, a = pl.pallas_call(
        _kf1_body,
        grid=(nb,),
        in_specs=[tok, tok128, _full((1, DIM)), _full((1, DIM)),
                  _full((MLPD, DIM)), _full((1, MLPD))],
        out_specs=[tok, tok2],
        out_shape=[jax.ShapeDtypeStruct((N, DIM), f32),
                   jax.ShapeDtypeStruct((N, MLPD), f32)],
    )(xt, y2u, r2(p['m_ln_g']), r2(p['m_ln_b']), p['m_fc1_w'],
      r2(p['m_fc1_b']))

    dwt = p['m_dw_w'].reshape(MLPD, 25).T
    ap = pl.BlockSpec((TB, MLPD), lambda i: (jnp.maximum(i - 1, 0), 0))
    an = pl.BlockSpec((TB, MLPD), lambda i: (jnp.minimum(i + 1, nb - 1), 0))
    out_tok = pl.pallas_call(
        functools.partial(_kf2_body, W=w),
        grid=(nb,),
        in_specs=[tok, ap, tok2, an, _full((25, MLPD)), _full((1, MLPD)),
                  _full((DIM, MLPD)), _full((1, DIM))],
        out_specs=fmaj,
        out_shape=jax.ShapeDtypeStruct((DIM, N), f32),
    )(xt2, a, a, a, dwt, r2(p['m_dw_b']), p['m_fc2_w'], r2(p['m_fc2_b']))

    return out_tok.reshape(b, DIM, h, w)


# speedup vs baseline: 2.1052x; 1.0150x over previous
import functools

import jax
import jax.numpy as jnp
import numpy as np
from jax.experimental import pallas as pl
from jax.experimental.pallas import tpu as pltpu

DIM = 96
QK = 96
MLPD = 192
HEADS = 4
NT = 8
GSZ = 128
DQ = QK // HEADS
TB = 512
NEG = -1e30


def _ln(x, g, b, eps=1e-5):
    m = jnp.mean(x, axis=-1, keepdims=True)
    v = jnp.mean((x - m) ** 2, axis=-1, keepdims=True)
    return (x - m) / jnp.sqrt(v + eps) * g + b


def _l2n(x, eps=1e-12):
    n = jnp.sqrt(jnp.sum(x * x, axis=-1, keepdims=True))
    return x / jnp.maximum(n, eps)


def _erf(z):
    s = jnp.sign(z)
    a = jnp.abs(z)
    t = 1.0 / (1.0 + 0.3275911 * a)
    poly = ((((1.061405429 * t - 1.453152027) * t + 1.421413741) * t
             - 0.284496736) * t + 0.254829592) * t
    return s * (1.0 - poly * jnp.exp(-a * a))


def _gelu(x):
    return 0.5 * x * (1.0 + _erf(x * 0.7071067811865476))


def _smax(s):
    m = jnp.max(s, axis=-1, keepdims=True)
    e = jnp.exp(s - m)
    return e / jnp.sum(e, axis=-1, keepdims=True)


def _dotT(a, b):
    return jax.lax.dot_general(a, b, (((1,), (1,)), ((), ())),
                               preferred_element_type=jnp.float32)


def _dot0(a, b):
    return jax.lax.dot_general(a, b, (((0,), (0,)), ((), ())),
                               preferred_element_type=jnp.float32)


def _full(shape):
    return pl.BlockSpec(shape, lambda *_: tuple(0 for _ in shape))


_SC_NC = 2
_SC_NS = 16
_SC_NW = _SC_NC * _SC_NS


def _sc_gather3(q, k, v, idx, n_rows):
    from jax.experimental.pallas import tpu_sc as plsc
    bpw = n_rows // _SC_NW
    nch, tail = divmod(bpw, 128)
    mesh = plsc.VectorSubcoreMesh(core_axis_name="c", subcore_axis_name="s")
    f32 = jnp.float32
    scratch = [pltpu.VMEM((128,), jnp.int32)] \
        + [pltpu.VMEM((128, 128), f32)] * 3 + [pltpu.SemaphoreType.DMA]
    if tail:
        scratch += [pltpu.VMEM((tail,), jnp.int32)] \
            + [pltpu.VMEM((tail, 128), f32)] * 3

    @functools.partial(
        pl.kernel, mesh=mesh,
        out_type=[jax.ShapeDtypeStruct((n_rows, 128), f32)] * 3,
        scratch_types=scratch)
    def go(q_hbm, k_hbm, v_hbm, idx_hbm, qo, ko, vo, idx_v, b0, b1, b2,
           sem, *tail_bufs):
        wid = jax.lax.axis_index("s") * _SC_NC + jax.lax.axis_index("c")
        base = wid * bpw

        def move(off, iv, bufs, nidx):
            pltpu.sync_copy(idx_hbm.at[pl.ds(off, nidx)], iv)
            hs = [pltpu.async_copy(t_hbm.at[iv], bv, sem)
                  for t_hbm, bv in ((q_hbm, bufs[0]), (k_hbm, bufs[1]),
                                    (v_hbm, bufs[2]))]
            for hh in hs:
                hh.wait()
            for o_hbm, bv in ((qo, bufs[0]), (ko, bufs[1]), (vo, bufs[2])):
                pltpu.sync_copy(bv, o_hbm.at[pl.ds(off, nidx)])

        for j in range(nch):
            move(base + j * 128, idx_v, (b0, b1, b2), 128)
        if tail:
            move(base + nch * 128, tail_bufs[0], tail_bufs[1:4], tail)

    return go(q, k, v, idx)


def _sc_scatter(y, idx, n_rows):
    from jax.experimental.pallas import tpu_sc as plsc
    bpw = n_rows // _SC_NW
    nch, tail = divmod(bpw, 128)
    mesh = plsc.VectorSubcoreMesh(core_axis_name="c", subcore_axis_name="s")
    f32 = jnp.float32
    scratch = [pltpu.VMEM((128,), jnp.int32), pltpu.VMEM((128, 128), f32),
               pltpu.SemaphoreType.DMA]
    if tail:
        scratch += [pltpu.VMEM((tail,), jnp.int32),
                    pltpu.VMEM((tail, 128), f32)]

    @functools.partial(
        pl.kernel, mesh=mesh,
        out_type=jax.ShapeDtypeStruct((n_rows, 128), f32),
        scratch_types=scratch)
    def go(y_hbm, idx_hbm, out_hbm, idx_v, buf, sem, *tail_bufs):
        wid = jax.lax.axis_index("s") * _SC_NC + jax.lax.axis_index("c")
        base = wid * bpw

        def move(off, iv, bv, nidx):
            pltpu.sync_copy(idx_hbm.at[pl.ds(off, nidx)], iv)
            pltpu.sync_copy(y_hbm.at[pl.ds(off, nidx)], bv)
            pltpu.async_copy(bv, out_hbm.at[iv], sem).wait()

        for j in range(nch):
            move(base + j * 128, idx_v, buf, 128)
        if tail:
            move(base + nch * 128, tail_bufs[0], tail_bufs[1], tail)

    return go(y, idx)


_SR = 512
_SL = 128
_SN = _SR * _SL


def _lex_lt(ka, va, kb, vb):
    return (ka < kb) | ((ka == kb) & (va < vb))


def _bitonic_body(k_ref, v_ref, ko_ref, vo_ref):
    keys = k_ref[...]
    vals = v_ref[...]
    row = jax.lax.broadcasted_iota(jnp.int32, (_SR, 1), 0)
    lane = jax.lax.broadcasted_iota(jnp.int32, (1, _SL), 1)

    kk = 2
    while kk <= _SN:
        j = kk // 2
        while j >= 1:
            if j >= _SL:
                jr = j // _SL
                g = _SR // (2 * jr)
                k4 = keys.reshape(g, 2, jr, _SL)
                pk = jnp.concatenate([k4[:, 1:2], k4[:, 0:1]], axis=1)
                pk = pk.reshape(_SR, _SL)
                v4 = vals.reshape(g, 2, jr, _SL)
                pv = jnp.concatenate([v4[:, 1:2], v4[:, 0:1]], axis=1)
                pv = pv.reshape(_SR, _SL)
                upper = (row & jr) == 0
            else:
                upper = (lane & j) == 0
                pk = jnp.where(
                    upper,
                    jnp.concatenate([keys[:, j:], keys[:, :j]], axis=1),
                    jnp.concatenate([keys[:, _SL - j:], keys[:, :_SL - j]],
                                    axis=1))
                pv = jnp.where(
                    upper,
                    jnp.concatenate([vals[:, j:], vals[:, :j]], axis=1),
                    jnp.concatenate([vals[:, _SL - j:], vals[:, :_SL - j]],
                                    axis=1))
            bk = kk.bit_length() - 1
            if bk >= 16:
                asc = jnp.full((1, 1), True)
            elif bk >= 7:
                asc = ((row >> (bk - 7)) & 1) == 0
            else:
                asc = (lane & kk) == 0
            s = _lex_lt(pk, pv, keys, vals)
            take = (s != upper) != asc
            keys = jnp.where(take, pk, keys)
            vals = jnp.where(take, pv, vals)
            j //= 2
        kk *= 2
    ko_ref[...] = keys
    vo_ref[...] = vals


def _tc_argsort(skey, n):
    pad = jnp.full((_SN - n,), 1e9, jnp.float32)
    keys2d = jnp.concatenate([skey, pad]).reshape(_SR, _SL)
    vals2d = jnp.arange(_SN, dtype=jnp.float32).reshape(_SR, _SL)
    _, vo = pl.pallas_call(
        _bitonic_body,
        out_shape=[jax.ShapeDtypeStruct((_SR, _SL), jnp.float32)] * 2,
    )(keys2d, vals2d)
    return vo.reshape(-1)[:n].astype(jnp.int32)


def _ka_body(xf_ref, ng_ref, nb_ref, eg_ref, eb_ref, ew_ref, ebias_ref,
             aw_ref, xt_ref, xn_ref, em_ref, ps_ref):
    i = pl.program_id(0)
    xt = jnp.transpose(xf_ref[...])
    xt_ref[...] = xt
    xn = _ln(xt, ng_ref[...], nb_ref[...])
    e0 = _ln(xn, eg_ref[...], eb_ref[...])
    embed = _gelu(_dotT(e0, ew_ref[...]) + ebias_ref[...])
    asg = _smax(_dotT(embed, aw_ref[...]))
    xn_ref[...] = xn
    em_ref[...] = embed

    @pl.when(i == 0)
    def _():
        ps_ref[...] = jnp.zeros_like(ps_ref)

    xnx = jnp.concatenate([xn, jnp.ones((TB, 32), jnp.float32)], axis=1)
    ps_ref[...] += _dot0(asg, xnx)


def _kt1_body(ps_ref, pg_ref, pb_ref, pq_ref, rq_ref, pre_ref, qp_ref):
    ps = ps_ref[...]
    content = ps[:, :DIM]
    weight = jnp.clip(ps[:, DIM:DIM + 1], 1e-6, None)
    protos = _l2n(_ln(content / weight, pg_ref[...], pb_ref[...]))
    pre_ref[...] = protos
    qp_ref[...] = _dotT(protos + pq_ref[...], rq_ref[...])


def _kb_body(xn_ref, em_ref, qp_ref, rk_ref, rv_ref, out_ref,
             m_ref, l_ref, acc_ref):
    i = pl.program_id(0)
    nb = pl.num_programs(0)

    @pl.when(i == 0)
    def _():
        m_ref[...] = jnp.full_like(m_ref, NEG)
        l_ref[...] = jnp.zeros_like(l_ref)
        acc_ref[...] = jnp.zeros_like(acc_ref)

    scale = float(QK) ** (-0.5)
    kt = _dotT(em_ref[...], rk_ref[...])
    vt = _dotT(xn_ref[...], rv_ref[...])
    s = _dotT(qp_ref[...], kt) * scale
    m_old = m_ref[...]
    m_new = jnp.maximum(m_old, jnp.max(s, axis=-1, keepdims=True))
    corr = jnp.exp(m_old - m_new)
    p = jnp.exp(s - m_new)
    l_new = l_ref[...] * corr + jnp.sum(p, axis=-1, keepdims=True)
    acc_new = acc_ref[...] * corr + jnp.dot(
        p, vt, preferred_element_type=jnp.float32)
    m_ref[...] = m_new
    l_ref[...] = l_new
    acc_ref[...] = acc_new

    @pl.when(i == nb - 1)
    def _():
        out_ref[...] = acc_new / l_new


def _kt2_body(pre_ref, rf_ref, gate_ref, pg_ref, pb_ref, pp_ref,
              pk_ref, pv_ref, c1_ref, pj_ref,
              pf_ref, kg_ref, vg_ref, wc_ref):
    gamma = jax.nn.sigmoid(gate_ref[...])
    pt = _l2n(_ln(pre_ref[...] + gamma * rf_ref[...],
                  pg_ref[...], pb_ref[...]))
    pf_ref[...] = _l2n(_dotT(pt, pp_ref[...]))
    kg_ref[...] = _dotT(pt, pk_ref[...])
    vg_ref[...] = _dotT(pt, pv_ref[...])
    wc = jnp.dot(c1_ref[...], pj_ref[...],
                 preferred_element_type=jnp.float32)
    wc = jnp.concatenate([wc, jnp.zeros((DIM, 32), jnp.float32)], axis=1)
    wc_ref[...] = jnp.concatenate(
        [wc, jnp.zeros((32, 128), jnp.float32)], axis=0)


def _kc_body(xn_ref, em_ref, pf_ref, tp_ref, qw_ref, kw_ref, vw_ref,
             sk_ref, q_ref, k_ref, v_ref):
    scale = float(QK) ** (-0.5)
    xn = xn_ref[...]
    tf = _l2n(_dotT(em_ref[...], tp_ref[...]))
    sm = _smax(_dotT(tf, pf_ref[...]) * scale)
    xs = jnp.max(sm, axis=-1, keepdims=True)
    iota8 = jax.lax.broadcasted_iota(jnp.int32, (TB, NT), 1)
    bel = jnp.min(jnp.where(sm >= xs, iota8, NT), axis=-1, keepdims=True)
    sk_ref[...] = bel.astype(jnp.float32) + 0.5 * (1.0 - xs)
    q_ref[...] = _dotT(xn, qw_ref[...])
    k_ref[...] = _dotT(xn, kw_ref[...])
    v_ref[...] = _dotT(xn, vw_ref[...])


def _attn_body(q_ref, k1_ref, k2_ref, v1_ref, v2_ref, kg_ref, vg_ref,
               wc_ref, o_ref):
    q = q_ref[...]
    k = jnp.concatenate([k1_ref[...], k2_ref[...]], axis=0)
    v = jnp.concatenate([v1_ref[...], v2_ref[...]], axis=0)
    kg = kg_ref[...]
    vg = vg_ref[...]
    lane = jax.lax.broadcasted_iota(jnp.int32, (1, 128), 1)
    scale = float(1.0 / np.sqrt(DQ))
    acc = jnp.zeros((GSZ, 128), jnp.float32)
    for h in range(HEADS):
        msk = (lane >= DQ * h) & (lane < DQ * (h + 1))
        km = jnp.where(msk, k, 0.0)
        p1 = _smax(_dotT(q, km) * scale)
        acc += jnp.dot(p1, jnp.where(msk, v, 0.0),
                       preferred_element_type=jnp.float32)
        p2 = _smax(_dotT(q, jnp.where(msk, kg, 0.0)) * scale)
        acc += jnp.dot(p2, jnp.where(msk, vg, 0.0),
                       preferred_element_type=jnp.float32)
    o_ref[...] = _dotT(acc, wc_ref[...])


def _kf1_body(xt_ref, y_ref, lg_ref, lb_ref, w1_ref, b1_ref,
              xt2_ref, a_ref):
    xt2 = xt_ref[...] + y_ref[:, :DIM]
    xt2_ref[...] = xt2
    z = _ln(xt2, lg_ref[...], lb_ref[...])
    a_ref[...] = _gelu(_dotT(z, w1_ref[...]) + b1_ref[...])


def _kf2_body(xt2_ref, ap_ref, ac_ref, an_ref, dwt_ref, db_ref,
              w2_ref, b2_ref, o_ref, W):
    g = pl.program_id(0)
    slab = jnp.concatenate([ap_ref[...], ac_ref[...], an_ref[...]], axis=0)
    nidx = (g * TB + jax.lax.broadcasted_iota(jnp.int32, (TB, 1), 0))
    hh = nidx // W
    ww = nidx - hh * W
    acc = jnp.zeros((TB, MLPD), jnp.float32)
    dwt = dwt_ref[...]
    for dx in range(-2, 3):
        wv = (ww + dx >= 0) & (ww + dx < W)
        part = jnp.zeros((TB, MLPD), jnp.float32)
        for dy in range(-2, 3):
            hv = (hh + dy >= 0) & (hh + dy < W)
            srow = TB + dy * W + dx
            sl = slab[srow:srow + TB, :]
            tap = (dy + 2) * 5 + (dx + 2)
            wt = dwt[tap:tap + 1, :]
            part += jnp.where(hv, sl, 0.0) * wt
        acc += jnp.where(wv, part, 0.0)
    a = ac_ref[...]
    zc = _gelu(acc + db_ref[...])
    z2 = a + zc
    res = xt2_ref[...] + _dotT(z2, w2_ref[...]) + b2_ref[...]
    o_ref[...] = jnp.transpose(res)


def kernel(x, params):
    p = params
    b, c, h, w = x.shape
    N = h * w
    assert b == 1 and N % TB == 0 and N % GSZ == 0
    nb = N // TB
    ng = N // GSZ
    f32 = jnp.float32

    def r2(v):
        return v.reshape(1, -1)

    def pad32(w):
        return jnp.concatenate(
            [w, jnp.zeros((32, w.shape[1]), f32)], axis=0)

    xf = x.reshape(c, N)

    tok = pl.BlockSpec((TB, DIM), lambda i: (i, 0))
    tok2 = pl.BlockSpec((TB, MLPD), lambda i: (i, 0))
    fmaj = pl.BlockSpec((DIM, TB), lambda i: (0, i))

    xt, xn, embed, psum = pl.pallas_call(
        _ka_body,
        grid=(nb,),
        in_specs=[fmaj, _full((1, DIM)), _full((1, DIM)), _full((1, DIM)),
                  _full((1, DIM)), _full((DIM, DIM)), _full((1, DIM)),
                  _full((NT, DIM))],
        out_specs=[tok, tok, tok, pl.BlockSpec((NT, 128), lambda i: (0, 0))],
        out_shape=[jax.ShapeDtypeStruct((N, DIM), f32),
                   jax.ShapeDtypeStruct((N, DIM), f32),
                   jax.ShapeDtypeStruct((N, DIM), f32),
                   jax.ShapeDtypeStruct((NT, 128), f32)],
    )(xf, r2(p['norm_g']), r2(p['norm_b']), r2(p['d_eln_g']),
      r2(p['d_eln_b']), p['d_embed_w'], r2(p['d_embed_b']), p['d_assign_w'])

    pre, qp = pl.pallas_call(
        _kt1_body,
        in_specs=[_full((NT, 128)), _full((1, DIM)), _full((1, DIM)),
                  _full((NT, DIM)), _full((DIM, DIM))],
        out_specs=[_full((NT, DIM)), _full((NT, DIM))],
        out_shape=[jax.ShapeDtypeStruct((NT, DIM), f32),
                   jax.ShapeDtypeStruct((NT, DIM), f32)],
    )(psum, r2(p['d_pn_g']), r2(p['d_pn_b']), p['d_protoq'], p['d_rq_w'])

    refine = pl.pallas_call(
        _kb_body,
        grid=(nb,),
        in_specs=[tok, tok, _full((NT, DIM)), _full((DIM, DIM)),
                  _full((DIM, DIM))],
        out_specs=pl.BlockSpec((NT, DIM), lambda i: (0, 0)),
        out_shape=jax.ShapeDtypeStruct((NT, DIM), f32),
        scratch_shapes=[pltpu.VMEM((NT, 1), f32), pltpu.VMEM((NT, 1), f32),
                        pltpu.VMEM((NT, DIM), f32)],
    )(xn, embed, qp, p['d_rk_w'], p['d_rv_w'])

    pf, kg, vg, wc = pl.pallas_call(
        _kt2_body,
        in_specs=[_full((NT, DIM)), _full((NT, DIM)), _full((1, 1)),
                  _full((1, DIM)), _full((1, DIM)), _full((DIM, DIM)),
                  _full((128, DIM)), _full((128, DIM)), _full((DIM, DIM)),
                  _full((DIM, DIM))],
        out_specs=[_full((NT, DIM)), _full((NT, 128)), _full((NT, 128)),
                   _full((128, 128))],
        out_shape=[jax.ShapeDtypeStruct((NT, DIM), f32),
                   jax.ShapeDtypeStruct((NT, 128), f32),
                   jax.ShapeDtypeStruct((NT, 128), f32),
                   jax.ShapeDtypeStruct((128, 128), f32)],
    )(pre, refine, p['d_gate'].reshape(1, 1), r2(p['d_pn_g']),
      r2(p['d_pn_b']), p['d_pp_w'], pad32(p['i_pk_w']),
      pad32(p['i_pv_w']), p['c1_w'], p['i_proj_w'])

    tok128 = pl.BlockSpec((TB, 128), lambda i: (i, 0))
    skey, q, k, v = pl.pallas_call(
        _kc_body,
        grid=(nb,),
        in_specs=[tok, tok, _full((NT, DIM)), _full((DIM, DIM)),
                  _full((128, DIM)), _full((128, DIM)), _full((128, DIM))],
        out_specs=[pl.BlockSpec((TB, 1), lambda i: (i, 0)),
                   tok128, tok128, tok128],
        out_shape=[jax.ShapeDtypeStruct((N, 1), f32)]
        + [jax.ShapeDtypeStruct((N, 128), f32)] * 3,
    )(xn, embed, pf, p['d_tp_w'], pad32(p['i_q_w']), pad32(p['i_k_w']),
      pad32(p['i_v_w']))

    sorted_idx = _tc_argsort(skey[:, 0], N)
    ext = N + GSZ
    n_rows = _SC_NW * (-(-(ext // _SC_NW) // 8) * 8)
    pad = n_rows - ext
    idx_ext = jnp.concatenate(
        [sorted_idx, sorted_idx[N - GSZ:][::-1],
         jnp.arange(pad, dtype=jnp.int32)])
    qg, k_ext, v_ext = _sc_gather3(q, k, v, idx_ext, n_rows)

    spec_q = pl.BlockSpec((GSZ, 128), lambda g: (g, 0))
    spec_kv1 = pl.BlockSpec((GSZ, 128), lambda g: (g, 0))
    spec_kv2 = pl.BlockSpec((GSZ, 128), lambda g: (g + 1, 0))
    spec_p = pl.BlockSpec((NT, 128), lambda g: (0, 0))
    y2 = pl.pallas_call(
        _attn_body,
        grid=(ng,),
        in_specs=[spec_q, spec_kv1, spec_kv2, spec_kv1, spec_kv2,
                  spec_p, spec_p, _full((128, 128))],
        out_specs=pl.BlockSpec((GSZ, 128), lambda g: (g, 0)),
        out_shape=jax.ShapeDtypeStruct((N, 128), f32),
    )(qg, k_ext, k_ext, v_ext, v_ext, kg, vg, wc)

    y2u = _sc_scatter(y2, sorted_idx, N)

    xt2, a = pl.pallas_call(
        _kf1_body,
        grid=(nb,),
        in_specs=[tok, tok128, _full((1, DIM)), _full((1, DIM)),
                  _full((MLPD, DIM)), _full((1, MLPD))],
        out_specs=[tok, tok2],
        out_shape=[jax.ShapeDtypeStruct((N, DIM), f32),
                   jax.ShapeDtypeStruct((N, MLPD), f32)],
    )(xt, y2u, r2(p['m_ln_g']), r2(p['m_ln_b']), p['m_fc1_w'],
      r2(p['m_fc1_b']))

    dwt = p['m_dw_w'].reshape(MLPD, 25).T
    ap = pl.BlockSpec((TB, MLPD), lambda i: (jnp.maximum(i - 1, 0), 0))
    an = pl.BlockSpec((TB, MLPD), lambda i: (jnp.minimum(i + 1, nb - 1), 0))
    out_tok = pl.pallas_call(
        functools.partial(_kf2_body, W=w),
        grid=(nb,),
        in_specs=[tok, ap, tok2, an, _full((25, MLPD)), _full((1, MLPD)),
                  _full((DIM, MLPD)), _full((1, DIM))],
        out_specs=fmaj,
        out_shape=jax.ShapeDtypeStruct((DIM, N), f32),
    )(xt2, a, a, a, dwt, r2(p['m_dw_b']), p['m_fc2_w'], r2(p['m_fc2_b']))

    return out_tok.reshape(b, DIM, h, w)
